# Initial kernel scaffold; baseline (speedup 1.0000x reference)
import jax
import jax.numpy as jnp
from jax.experimental import pallas as pl


def kernel(x, scores, selfscores, ctx_ids, prev_x_tokens, W_gen, b_gen, W1, b1, W2, b2):
    raise NotImplementedError("write your pallas kernel here")


# speedup vs baseline: 1.2420x; 1.2420x over previous
import functools

import jax
import jax.numpy as jnp
from jax import lax
from jax.experimental import pallas as pl
from jax.experimental.pallas import tpu as pltpu
from jax.experimental.pallas import tpu_sc as plsc

B = 1024
V = 100000
D = 128
H = 128
L = 200
T = 50
K = 256
TV = 2048
NV = (V + TV - 1) // TV
R = 8
NC = 2
NS = 16
NW = NC * NS
RPW = B // NW
CH = RPW * K // 128


def _prep_body(x_ref, w1_ref, b1_ref, w2_ref, b2_ref, sc_ref, ssc_ref,
               idx_ref, sval_ref, mix0_ref):
    xc = x_ref[...]
    r = jnp.tanh(jnp.dot(xc, w1_ref[...], preferred_element_type=jnp.float32)
                 + b1_ref[...])
    g = jnp.dot(r, w2_ref[...], preferred_element_type=jnp.float32) + b2_ref[...]
    g = g - jnp.max(g, axis=1, keepdims=True)
    eg = jnp.exp(g)
    mix = eg / jnp.sum(eg, axis=1, keepdims=True)
    mix0_ref[...] = mix[:, 0:1]

    s = sc_ref[...]
    s = s - jnp.max(s, axis=1, keepdims=True)
    es = jnp.exp(s)
    al = es / jnp.sum(es, axis=1, keepdims=True)

    ss = ssc_ref[...]
    ss = ss - jnp.max(ss, axis=1, keepdims=True)
    ess = jnp.exp(ss)
    sal = ess / jnp.sum(ess, axis=1, keepdims=True)

    val = jnp.concatenate(
        [al * mix[:, 1:2], sal * mix[:, 2:3],
         jnp.zeros((R, K - L - (T - 1)), jnp.float32)], axis=1)

    idx = idx_ref[...]
    eq = idx[:, :, None] == idx[:, None, :]
    sval_ref[...] = jnp.sum(jnp.where(eq, val[:, None, :], 0.0), axis=2)


def _stats_body(xb_ref, wb_ref, bg_ref, m_ref, s_ref):
    j = pl.program_id(0)

    @pl.when(j == 0)
    def _():
        m_ref[...] = jnp.full((B, 1), -1e30, jnp.float32)
        s_ref[...] = jnp.zeros((B, 1), jnp.float32)

    l = jnp.dot(xb_ref[...], wb_ref[...],
                preferred_element_type=jnp.float32) + bg_ref[...]
    col = lax.broadcasted_iota(jnp.int32, (1, TV), 1) + j * TV
    l = jnp.where(col < V, l, -1e30)
    m_old = m_ref[...]
    m_new = jnp.maximum(m_old, jnp.max(l, axis=1, keepdims=True))
    p = jnp.exp(l - m_new)
    s_ref[...] = s_ref[...] * jnp.exp(m_old - m_new) + jnp.sum(
        p, axis=1, keepdims=True)
    m_ref[...] = m_new


def _emit_body(xb_ref, wb_ref, bg_ref, m_ref, s_ref, mix0_ref, o_ref):
    l = jnp.dot(xb_ref[...], wb_ref[...],
                preferred_element_type=jnp.float32) + bg_ref[...]
    o_ref[...] = jnp.exp(l - m_ref[...]) * (mix0_ref[...] / s_ref[...])


def _sc_body(out_hbm, fidx_hbm, sval_hbm, idx_v, sv_v, g_v, sem):
    c = lax.axis_index("c")
    s = lax.axis_index("s")
    wid = s * NC + c
    pltpu.sync_copy(fidx_hbm.at[wid], idx_v)
    pltpu.sync_copy(sval_hbm.at[wid], sv_v)

    def fire_gather(j, carry):
        pltpu.make_async_copy(out_hbm.at[idx_v.at[j]], g_v.at[j], sem).start()
        return carry

    def drain_gather(j, carry):
        pltpu.make_async_copy(out_hbm.at[idx_v.at[j]], g_v.at[j], sem).wait()
        return carry

    def addrow(j, carry):
        for t in range(128 // 16):
            sl = pl.ds(t * 16, 16)
            g_v[j, sl] = g_v[j, sl] + sv_v[j, sl]
        return carry

    def fire_scatter(j, carry):
        pltpu.make_async_copy(g_v.at[j], out_hbm.at[idx_v.at[j]], sem).start()
        return carry

    def drain_scatter(j, carry):
        pltpu.make_async_copy(g_v.at[j], out_hbm.at[idx_v.at[j]], sem).wait()
        return carry

    lax.fori_loop(0, CH, fire_gather, 0)
    lax.fori_loop(0, CH, drain_gather, 0)
    lax.fori_loop(0, CH, addrow, 0)
    lax.fori_loop(0, CH, fire_scatter, 0)
    lax.fori_loop(0, CH, drain_scatter, 0)


def kernel(x, scores, selfscores, ctx_ids, prev_x_tokens,
           W_gen, b_gen, W1, b1, W2, b2):
    idx_all = jnp.concatenate([ctx_ids, prev_x_tokens[:, :-1]], axis=1)
    idx_pad = jnp.pad(idx_all, ((0, 0), (0, K - idx_all.shape[1])))
    fidx = (idx_pad + jnp.arange(B, dtype=jnp.int32)[:, None] * V)
    fidx = fidx.reshape(NW, CH, 128)
    xb = x.astype(jnp.bfloat16)
    wb = W_gen.astype(jnp.bfloat16)
    bg2 = b_gen.reshape(1, V)
    b1_2 = b1.reshape(1, H)
    b2_2 = b2.reshape(1, 3)

    sval, mix0 = pl.pallas_call(
        _prep_body,
        grid=(B // R,),
        in_specs=[
            pl.BlockSpec((R, D), lambda i: (i, 0)),
            pl.BlockSpec((D, H), lambda i: (0, 0)),
            pl.BlockSpec((1, H), lambda i: (0, 0)),
            pl.BlockSpec((H, 3), lambda i: (0, 0)),
            pl.BlockSpec((1, 3), lambda i: (0, 0)),
            pl.BlockSpec((R, L), lambda i: (i, 0)),
            pl.BlockSpec((R, T - 1), lambda i: (i, 0)),
            pl.BlockSpec((R, K), lambda i: (i, 0)),
        ],
        out_specs=[
            pl.BlockSpec((R, K), lambda i: (i, 0)),
            pl.BlockSpec((R, 1), lambda i: (i, 0)),
        ],
        out_shape=[
            jax.ShapeDtypeStruct((B, K), jnp.float32),
            jax.ShapeDtypeStruct((B, 1), jnp.float32),
        ],
        name="ptrgen_prep",
    )(x, W1, b1_2, W2, b2_2, scores, selfscores, idx_pad)

    m, ssum = pl.pallas_call(
        _stats_body,
        grid=(NV,),
        in_specs=[
            pl.BlockSpec((B, D), lambda j: (0, 0)),
            pl.BlockSpec((D, TV), lambda j: (0, j)),
            pl.BlockSpec((1, TV), lambda j: (0, j)),
        ],
        out_specs=[
            pl.BlockSpec((B, 1), lambda j: (0, 0)),
            pl.BlockSpec((B, 1), lambda j: (0, 0)),
        ],
        out_shape=[
            jax.ShapeDtypeStruct((B, 1), jnp.float32),
            jax.ShapeDtypeStruct((B, 1), jnp.float32),
        ],
        name="ptrgen_stats",
    )(xb, wb, bg2)

    out_dense = pl.pallas_call(
        _emit_body,
        grid=(NV,),
        in_specs=[
            pl.BlockSpec((B, D), lambda j: (0, 0)),
            pl.BlockSpec((D, TV), lambda j: (0, j)),
            pl.BlockSpec((1, TV), lambda j: (0, j)),
            pl.BlockSpec((B, 1), lambda j: (0, 0)),
            pl.BlockSpec((B, 1), lambda j: (0, 0)),
            pl.BlockSpec((B, 1), lambda j: (0, 0)),
        ],
        out_specs=pl.BlockSpec((B, TV), lambda j: (0, j)),
        out_shape=jax.ShapeDtypeStruct((B, V), jnp.float32),
        name="ptrgen_emit",
    )(xb, wb, bg2, m, ssum, mix0)

    sval_w = sval.reshape(NW, CH, 128)
    sc_scatter = pl.kernel(
        _sc_body,
        out_type=(),
        mesh=plsc.VectorSubcoreMesh(core_axis_name="c", subcore_axis_name="s"),
        scratch_types=[
            pltpu.VMEM((CH, 128), jnp.int32),
            pltpu.VMEM((CH, 128), jnp.float32),
            pltpu.VMEM((CH, 128), jnp.float32),
            pltpu.SemaphoreType.DMA,
        ],
        name="ptrgen_scatter",
    )
    out_ref = jax.new_ref(out_dense.reshape(B * V))
    sc_scatter(out_ref, fidx, sval_w)
    return out_ref[...].reshape(B, V)


# speedup vs baseline: 3.0499x; 2.4557x over previous
import jax
import jax.numpy as jnp
from jax import lax
from jax.experimental import pallas as pl
from jax.experimental.pallas import tpu as pltpu
from jax.experimental.pallas import tpu_sc as plsc

B = 1024
V = 100000
D = 128
H = 128
L = 200
T = 50
K = 256
TV = 2048
TVd8 = TV // 8
NV = (V + TV - 1) // TV
R = 8
NC = 2
NS = 16
NW = NC * NS
RPW = B // NW
CH = RPW * K // 128


def _prep_body(x_ref, w1_ref, b1_ref, w2_ref, b2_ref, sc_ref, ssc_ref,
               idx_ref, sval_ref, mix0_ref):
    xc = x_ref[...]
    r = jnp.tanh(jnp.dot(xc, w1_ref[...], preferred_element_type=jnp.float32)
                 + b1_ref[...])
    g = jnp.dot(r, w2_ref[...], preferred_element_type=jnp.float32) + b2_ref[...]
    g = g - jnp.max(g, axis=1, keepdims=True)
    eg = jnp.exp(g)
    mix = eg / jnp.sum(eg, axis=1, keepdims=True)
    mix0_ref[...] = mix[:, 0:1]

    s = sc_ref[...]
    s = s - jnp.max(s, axis=1, keepdims=True)
    es = jnp.exp(s)
    al = es / jnp.sum(es, axis=1, keepdims=True)

    ss = ssc_ref[...]
    ss = ss - jnp.max(ss, axis=1, keepdims=True)
    ess = jnp.exp(ss)
    sal = ess / jnp.sum(ess, axis=1, keepdims=True)

    val = jnp.concatenate(
        [al * mix[:, 1:2], sal * mix[:, 2:3],
         jnp.zeros((R, K - L - (T - 1)), jnp.float32)], axis=1)

    idx = idx_ref[...]
    eq = idx[:, :, None] == idx[:, None, :]
    sval_ref[...] = jnp.sum(jnp.where(eq, val[:, None, :], 0.0), axis=2)


def _dotT(wb_tile, xbT):
    return lax.dot_general(wb_tile, xbT, (((0,), (0,)), ((), ())),
                           preferred_element_type=jnp.float32)


def _stats_body(xb_ref, wb_ref, bg_ref, m_ref, s_ref):
    j = pl.program_id(0)

    @pl.when(j == 0)
    def _():
        m_ref[...] = jnp.full((1, B), -1e30, jnp.float32)
        s_ref[...] = jnp.zeros((1, B), jnp.float32)

    l = _dotT(wb_ref[...], xb_ref[...]) + bg_ref[...]
    row = lax.broadcasted_iota(jnp.int32, (TV, 1), 0)
    l = jnp.where(row < V - j * TV, l, -1e30)
    m_old = m_ref[...]
    m_new = jnp.maximum(m_old, jnp.max(l, axis=0, keepdims=True))
    p = jnp.exp(l - m_new)
    s_ref[...] = s_ref[...] * jnp.exp(m_old - m_new) + jnp.sum(
        p, axis=0, keepdims=True)
    m_ref[...] = m_new


def _emit_body(xb_ref, wb_ref, bg_ref, m_ref, s_ref, mix0_ref, o_ref):
    l = _dotT(wb_ref[...], xb_ref[...]) + bg_ref[...]
    p = jnp.exp(l - m_ref[...]) * (mix0_ref[...] / s_ref[...])
    for bt in range(B // 128):
        o_ref[:, bt, :, :] = p[:, bt * 128:(bt + 1) * 128].reshape(TVd8, 8, 128)


def _sc_body(out_hbm, fidx_hbm, sval_hbm, idx_v, sv_v, g_v, semg, sems):
    c = lax.axis_index("c")
    s = lax.axis_index("s")
    wid = s * NC + c
    pltpu.sync_copy(fidx_hbm.at[wid], idx_v)
    pltpu.sync_copy(sval_hbm.at[wid], sv_v)

    def fire_gather(j, carry):
        pltpu.make_async_copy(out_hbm.at[idx_v.at[j]], g_v.at[j], semg).start()
        return carry

    def drain_add(j, carry):
        pltpu.make_async_copy(out_hbm.at[idx_v.at[j]], g_v.at[j], semg).wait()
        for t in range(128 // 16):
            sl = pl.ds(t * 16, 16)
            g_v[j, sl] = g_v[j, sl] + sv_v[j, sl]
        return carry

    def fire_scatter(j, carry):
        pltpu.make_async_copy(g_v.at[j], out_hbm.at[idx_v.at[j]], sems).start()
        return carry

    def drain_scatter(j, carry):
        pltpu.make_async_copy(g_v.at[j], out_hbm.at[idx_v.at[j]], sems).wait()
        return carry

    lax.fori_loop(0, CH, fire_gather, 0)
    lax.fori_loop(0, CH, drain_add, 0)
    lax.fori_loop(0, CH, fire_scatter, 0)
    lax.fori_loop(0, CH, drain_scatter, 0)


def kernel(x, scores, selfscores, ctx_ids, prev_x_tokens,
           W_gen, b_gen, W1, b1, W2, b2):
    idx_all = jnp.concatenate([ctx_ids, prev_x_tokens[:, :-1]], axis=1)
    idx_pad = jnp.pad(idx_all, ((0, 0), (0, K - idx_all.shape[1])))
    brow = jnp.arange(B, dtype=jnp.int32)[:, None]
    fidx = ((idx_pad >> 3) * 8192 + (brow >> 7) * 1024
            + (idx_pad & 7) * 128 + (brow & 127))
    fidx = fidx.reshape(NW, CH, 128)
    xbT = x.T.astype(jnp.bfloat16)
    wb = W_gen.astype(jnp.bfloat16)
    bgT = b_gen.reshape(V, 1)
    b1_2 = b1.reshape(1, H)
    b2_2 = b2.reshape(1, 3)

    sval, mix0 = pl.pallas_call(
        _prep_body,
        grid=(B // R,),
        in_specs=[
            pl.BlockSpec((R, D), lambda i: (i, 0)),
            pl.BlockSpec((D, H), lambda i: (0, 0)),
            pl.BlockSpec((1, H), lambda i: (0, 0)),
            pl.BlockSpec((H, 3), lambda i: (0, 0)),
            pl.BlockSpec((1, 3), lambda i: (0, 0)),
            pl.BlockSpec((R, L), lambda i: (i, 0)),
            pl.BlockSpec((R, T - 1), lambda i: (i, 0)),
            pl.BlockSpec((R, K), lambda i: (i, 0)),
        ],
        out_specs=[
            pl.BlockSpec((R, K), lambda i: (i, 0)),
            pl.BlockSpec((R, 1), lambda i: (i, 0)),
        ],
        out_shape=[
            jax.ShapeDtypeStruct((B, K), jnp.float32),
            jax.ShapeDtypeStruct((B, 1), jnp.float32),
        ],
        name="ptrgen_prep",
    )(x, W1, b1_2, W2, b2_2, scores, selfscores, idx_pad)
    mix0T = mix0.T

    m, ssum = pl.pallas_call(
        _stats_body,
        grid=(NV,),
        in_specs=[
            pl.BlockSpec((D, B), lambda j: (0, 0)),
            pl.BlockSpec((D, TV), lambda j: (0, j)),
            pl.BlockSpec((TV, 1), lambda j: (j, 0)),
        ],
        out_specs=[
            pl.BlockSpec((1, B), lambda j: (0, 0)),
            pl.BlockSpec((1, B), lambda j: (0, 0)),
        ],
        out_shape=[
            jax.ShapeDtypeStruct((1, B), jnp.float32),
            jax.ShapeDtypeStruct((1, B), jnp.float32),
        ],
        name="ptrgen_stats",
    )(xbT, wb, bgT)

    z = pl.pallas_call(
        _emit_body,
        grid=(NV,),
        in_specs=[
            pl.BlockSpec((D, B), lambda j: (0, 0)),
            pl.BlockSpec((D, TV), lambda j: (0, j)),
            pl.BlockSpec((TV, 1), lambda j: (j, 0)),
            pl.BlockSpec((1, B), lambda j: (0, 0)),
            pl.BlockSpec((1, B), lambda j: (0, 0)),
            pl.BlockSpec((1, B), lambda j: (0, 0)),
        ],
        out_specs=pl.BlockSpec((TVd8, 8, 8, 128), lambda j: (j, 0, 0, 0)),
        out_shape=jax.ShapeDtypeStruct((V // 8, 8, 8, 128), jnp.float32),
        name="ptrgen_emit",
    )(xbT, wb, bgT, m, ssum, mix0T)

    sval_w = sval.reshape(NW, CH, 128)
    sc_scatter = pl.kernel(
        _sc_body,
        out_type=(),
        mesh=plsc.VectorSubcoreMesh(core_axis_name="c", subcore_axis_name="s"),
        scratch_types=[
            pltpu.VMEM((CH, 128), jnp.int32),
            pltpu.VMEM((CH, 128), jnp.float32),
            pltpu.VMEM((CH, 128), jnp.float32),
            pltpu.SemaphoreType.DMA,
            pltpu.SemaphoreType.DMA,
        ],
        name="ptrgen_scatter",
    )
    out_ref = jax.new_ref(z.reshape(B * V))
    sc_scatter(out_ref, fidx, sval_w)
    zf = out_ref[...]
    return zf.reshape(V // 8, 8, 8, 128).transpose(1, 3, 0, 2).reshape(B, V)


# speedup vs baseline: 3.0507x; 1.0003x over previous
import jax
import jax.numpy as jnp
from jax import lax
from jax.experimental import pallas as pl
from jax.experimental.pallas import tpu as pltpu
from jax.experimental.pallas import tpu_sc as plsc

B = 1024
V = 100000
D = 128
H = 128
L = 200
T = 50
K = 256
TV = 2048
TVd8 = TV // 8
NV = (V + TV - 1) // TV
R = 8
NC = 2
NS = 16
NW = NC * NS
RPW = B // NW
NST = 8
SW = RPW * K // NST


def _prep_body(x_ref, w1_ref, b1_ref, w2_ref, b2_ref, sc_ref, ssc_ref,
               idx_ref, sval_ref, mix0_ref):
    xc = x_ref[...]
    r = jnp.tanh(jnp.dot(xc, w1_ref[...], preferred_element_type=jnp.float32)
                 + b1_ref[...])
    g = jnp.dot(r, w2_ref[...], preferred_element_type=jnp.float32) + b2_ref[...]
    g = g - jnp.max(g, axis=1, keepdims=True)
    eg = jnp.exp(g)
    mix = eg / jnp.sum(eg, axis=1, keepdims=True)
    mix0_ref[...] = mix[:, 0:1]

    s = sc_ref[...]
    s = s - jnp.max(s, axis=1, keepdims=True)
    es = jnp.exp(s)
    al = es / jnp.sum(es, axis=1, keepdims=True)

    ss = ssc_ref[...]
    ss = ss - jnp.max(ss, axis=1, keepdims=True)
    ess = jnp.exp(ss)
    sal = ess / jnp.sum(ess, axis=1, keepdims=True)

    val = jnp.concatenate(
        [al * mix[:, 1:2], sal * mix[:, 2:3],
         jnp.zeros((R, K - L - (T - 1)), jnp.float32)], axis=1)

    idx = idx_ref[...]
    eq = idx[:, :, None] == idx[:, None, :]
    sval_ref[...] = jnp.sum(jnp.where(eq, val[:, None, :], 0.0), axis=2)


def _dotT(wb_tile, xbT):
    return lax.dot_general(wb_tile, xbT, (((0,), (0,)), ((), ())),
                           preferred_element_type=jnp.float32)


def _stats_body(xb_ref, wb_ref, bg_ref, m_ref, s_ref):
    j = pl.program_id(0)

    @pl.when(j == 0)
    def _():
        m_ref[...] = jnp.full((1, B), -1e30, jnp.float32)
        s_ref[...] = jnp.zeros((1, B), jnp.float32)

    l = _dotT(wb_ref[...], xb_ref[...]) + bg_ref[...]
    row = lax.broadcasted_iota(jnp.int32, (TV, 1), 0)
    l = jnp.where(row < V - j * TV, l, -1e30)
    m_old = m_ref[...]
    m_new = jnp.maximum(m_old, jnp.max(l, axis=0, keepdims=True))
    p = jnp.exp(l - m_new)
    s_ref[...] = s_ref[...] * jnp.exp(m_old - m_new) + jnp.sum(
        p, axis=0, keepdims=True)
    m_ref[...] = m_new


def _emit_body(xb_ref, wb_ref, bg_ref, m_ref, s_ref, mix0_ref, o_ref):
    l = _dotT(wb_ref[...], xb_ref[...]) + bg_ref[...]
    p = jnp.exp(l - m_ref[...]) * (mix0_ref[...] / s_ref[...])
    for bt in range(B // 128):
        o_ref[:, bt, :, :] = p[:, bt * 128:(bt + 1) * 128].reshape(TVd8, 8, 128)


def _sc_body(out_hbm, fidx_hbm, sval_hbm, *rest):
    idx_b = rest[:NST]
    g_b = rest[NST:2 * NST]
    sv_v, semg, sems = rest[2 * NST:]
    c = lax.axis_index("c")
    s = lax.axis_index("s")
    wid = s * NC + c
    pltpu.sync_copy(sval_hbm.at[wid], sv_v)
    for j in range(NST):
        pltpu.sync_copy(fidx_hbm.at[wid, j], idx_b[j])
        pltpu.make_async_copy(out_hbm.at[idx_b[j]], g_b[j], semg).start()
    for j in range(NST):
        pltpu.make_async_copy(out_hbm.at[idx_b[j]], g_b[j], semg).wait()

        def addchunk(t, carry, j=j):
            sl = pl.ds(t * 16, 16)
            g_b[j][sl] = g_b[j][sl] + sv_v[j, sl]
            return carry

        lax.fori_loop(0, SW // 16, addchunk, 0)
        pltpu.make_async_copy(g_b[j], out_hbm.at[idx_b[j]], sems).start()
    for j in range(NST):
        pltpu.make_async_copy(g_b[j], out_hbm.at[idx_b[j]], sems).wait()


def kernel(x, scores, selfscores, ctx_ids, prev_x_tokens,
           W_gen, b_gen, W1, b1, W2, b2):
    idx_all = jnp.concatenate([ctx_ids, prev_x_tokens[:, :-1]], axis=1)
    idx_pad = jnp.pad(idx_all, ((0, 0), (0, K - idx_all.shape[1])))
    brow = jnp.arange(B, dtype=jnp.int32)[:, None]
    fidx = ((idx_pad >> 3) * 8192 + (brow >> 7) * 1024
            + (idx_pad & 7) * 128 + (brow & 127))
    fidx = fidx.reshape(NW, NST, SW)
    xbT = x.T.astype(jnp.bfloat16)
    wb = W_gen.astype(jnp.bfloat16)
    bgT = b_gen.reshape(V, 1)
    b1_2 = b1.reshape(1, H)
    b2_2 = b2.reshape(1, 3)

    sval, mix0 = pl.pallas_call(
        _prep_body,
        grid=(B // R,),
        in_specs=[
            pl.BlockSpec((R, D), lambda i: (i, 0)),
            pl.BlockSpec((D, H), lambda i: (0, 0)),
            pl.BlockSpec((1, H), lambda i: (0, 0)),
            pl.BlockSpec((H, 3), lambda i: (0, 0)),
            pl.BlockSpec((1, 3), lambda i: (0, 0)),
            pl.BlockSpec((R, L), lambda i: (i, 0)),
            pl.BlockSpec((R, T - 1), lambda i: (i, 0)),
            pl.BlockSpec((R, K), lambda i: (i, 0)),
        ],
        out_specs=[
            pl.BlockSpec((R, K), lambda i: (i, 0)),
            pl.BlockSpec((R, 1), lambda i: (i, 0)),
        ],
        out_shape=[
            jax.ShapeDtypeStruct((B, K), jnp.float32),
            jax.ShapeDtypeStruct((B, 1), jnp.float32),
        ],
        name="ptrgen_prep",
    )(x, W1, b1_2, W2, b2_2, scores, selfscores, idx_pad)
    mix0T = mix0.T

    m, ssum = pl.pallas_call(
        _stats_body,
        grid=(NV,),
        in_specs=[
            pl.BlockSpec((D, B), lambda j: (0, 0)),
            pl.BlockSpec((D, TV), lambda j: (0, j)),
            pl.BlockSpec((TV, 1), lambda j: (j, 0)),
        ],
        out_specs=[
            pl.BlockSpec((1, B), lambda j: (0, 0)),
            pl.BlockSpec((1, B), lambda j: (0, 0)),
        ],
        out_shape=[
            jax.ShapeDtypeStruct((1, B), jnp.float32),
            jax.ShapeDtypeStruct((1, B), jnp.float32),
        ],
        name="ptrgen_stats",
    )(xbT, wb, bgT)

    z = pl.pallas_call(
        _emit_body,
        grid=(NV,),
        in_specs=[
            pl.BlockSpec((D, B), lambda j: (0, 0)),
            pl.BlockSpec((D, TV), lambda j: (0, j)),
            pl.BlockSpec((TV, 1), lambda j: (j, 0)),
            pl.BlockSpec((1, B), lambda j: (0, 0)),
            pl.BlockSpec((1, B), lambda j: (0, 0)),
            pl.BlockSpec((1, B), lambda j: (0, 0)),
        ],
        out_specs=pl.BlockSpec((TVd8, 8, 8, 128), lambda j: (j, 0, 0, 0)),
        out_shape=jax.ShapeDtypeStruct((V // 8, 8, 8, 128), jnp.float32),
        name="ptrgen_emit",
    )(xbT, wb, bgT, m, ssum, mix0T)

    sval_w = sval.reshape(NW, NST, SW)
    sc_scatter = pl.kernel(
        _sc_body,
        out_type=(),
        mesh=plsc.VectorSubcoreMesh(core_axis_name="c", subcore_axis_name="s"),
        scratch_types=(
            [pltpu.VMEM((SW,), jnp.int32) for _ in range(NST)]
            + [pltpu.VMEM((SW,), jnp.float32) for _ in range(NST)]
            + [pltpu.VMEM((NST, SW), jnp.float32),
               pltpu.SemaphoreType.DMA,
               pltpu.SemaphoreType.DMA]
        ),
        name="ptrgen_scatter",
    )
    out_ref = jax.new_ref(z.reshape(B * V))
    sc_scatter(out_ref, fidx, sval_w)
    zf = out_ref[...]
    return zf.reshape(V // 8, 8, 8, 128).transpose(1, 3, 0, 2).reshape(B, V)


# speedup vs baseline: 3.5178x; 1.1531x over previous
import jax
import jax.numpy as jnp
from jax import lax
from jax.experimental import pallas as pl
from jax.experimental.pallas import tpu as pltpu
from jax.experimental.pallas import tpu_sc as plsc

B = 1024
V = 100000
D = 128
H = 128
L = 200
T = 50
K = 256
TV = 2048
TVd8 = TV // 8
NV = (V + TV - 1) // TV
TVS = 4096
VP = 102400
DA = D + 1
R = 32
NC = 2
NS = 16
NW = NC * NS
RPW = B // NW
NST = 8
SW = RPW * K // NST


def _prep_body(x_ref, w1_ref, b1_ref, w2_ref, b2_ref, sc_ref, ssc_ref,
               idx_ref, sval_ref, mix0_ref):
    xc = x_ref[...]
    r = jnp.tanh(jnp.dot(xc, w1_ref[...], preferred_element_type=jnp.float32)
                 + b1_ref[...])
    g = jnp.dot(r, w2_ref[...], preferred_element_type=jnp.float32) + b2_ref[...]
    g = g - jnp.max(g, axis=1, keepdims=True)
    eg = jnp.exp(g)
    mix = eg / jnp.sum(eg, axis=1, keepdims=True)
    mix0_ref[...] = mix[:, 0:1]

    s = sc_ref[...]
    s = s - jnp.max(s, axis=1, keepdims=True)
    es = jnp.exp(s)
    al = es / jnp.sum(es, axis=1, keepdims=True)

    ss = ssc_ref[...]
    ss = ss - jnp.max(ss, axis=1, keepdims=True)
    ess = jnp.exp(ss)
    sal = ess / jnp.sum(ess, axis=1, keepdims=True)

    val = jnp.concatenate(
        [al * mix[:, 1:2], sal * mix[:, 2:3],
         jnp.zeros((R, K - L - (T - 1)), jnp.float32)], axis=1)

    idx = idx_ref[...]
    eq = idx[:, :, None] == idx[:, None, :]
    sval_ref[...] = jnp.sum(jnp.where(eq, val[:, None, :], 0.0), axis=2)


def _dotT(wb_tile, xbT):
    return lax.dot_general(wb_tile, xbT, (((0,), (0,)), ((), ())),
                           preferred_element_type=jnp.float32)


def _stats_body(xb_ref, wb_ref, m_ref, s_ref):
    j = pl.program_id(0)

    @pl.when(j == 0)
    def _():
        m_ref[...] = jnp.full((1, B), -1e30, jnp.float32)
        s_ref[...] = jnp.zeros((1, B), jnp.float32)

    l = _dotT(wb_ref[...], xb_ref[...])
    m_old = m_ref[...]
    m_new = jnp.maximum(m_old, jnp.max(l, axis=0, keepdims=True))
    p = jnp.exp(l - m_new)
    s_ref[...] = s_ref[...] * jnp.exp(m_old - m_new) + jnp.sum(
        p, axis=0, keepdims=True)
    m_ref[...] = m_new


def _emit_body(xb_ref, wb_ref, m_ref, s_ref, mix0_ref, o_ref):
    l = _dotT(wb_ref[...], xb_ref[...])
    p = jnp.exp(l - m_ref[...]) * (mix0_ref[...] / s_ref[...])
    for bt in range(B // 128):
        o_ref[:, bt, :, :] = p[:, bt * 128:(bt + 1) * 128].reshape(TVd8, 8, 128)


def _sc_body(out_hbm, fidx_hbm, sval_hbm, *rest):
    idx_b = rest[:NST]
    g_b = rest[NST:2 * NST]
    sv_v, semg, sems = rest[2 * NST:]
    c = lax.axis_index("c")
    s = lax.axis_index("s")
    wid = s * NC + c
    pltpu.sync_copy(sval_hbm.at[wid], sv_v)
    for j in range(NST):
        pltpu.sync_copy(fidx_hbm.at[wid, j], idx_b[j])
        pltpu.make_async_copy(out_hbm.at[idx_b[j]], g_b[j], semg).start()
    for j in range(NST):
        pltpu.make_async_copy(out_hbm.at[idx_b[j]], g_b[j], semg).wait()

        def addchunk(t, carry, j=j):
            sl = pl.ds(t * 16, 16)
            g_b[j][sl] = g_b[j][sl] + sv_v[j, sl]
            return carry

        lax.fori_loop(0, SW // 16, addchunk, 0)
        pltpu.make_async_copy(g_b[j], out_hbm.at[idx_b[j]], sems).start()
    for j in range(NST):
        pltpu.make_async_copy(g_b[j], out_hbm.at[idx_b[j]], sems).wait()


def kernel(x, scores, selfscores, ctx_ids, prev_x_tokens,
           W_gen, b_gen, W1, b1, W2, b2):
    idx_all = jnp.concatenate([ctx_ids, prev_x_tokens[:, :-1]], axis=1)
    idx_pad = jnp.pad(idx_all, ((0, 0), (0, K - idx_all.shape[1])))
    brow = jnp.arange(B, dtype=jnp.int32)[:, None]
    fidx = ((idx_pad >> 3) * 8192 + (brow >> 7) * 1024
            + (idx_pad & 7) * 128 + (brow & 127))
    fidx = fidx.reshape(NW, NST, SW)
    xbT = jnp.concatenate(
        [x.T, jnp.ones((1, B), x.dtype)], axis=0).astype(jnp.bfloat16)
    wb = jnp.concatenate([
        jnp.pad(W_gen, ((0, 0), (0, VP - V))),
        jnp.pad(b_gen, (0, VP - V), constant_values=-1e30)[None, :],
    ], axis=0).astype(jnp.bfloat16)
    b1_2 = b1.reshape(1, H)
    b2_2 = b2.reshape(1, 3)

    sval, mix0 = pl.pallas_call(
        _prep_body,
        grid=(B // R,),
        in_specs=[
            pl.BlockSpec((R, D), lambda i: (i, 0)),
            pl.BlockSpec((D, H), lambda i: (0, 0)),
            pl.BlockSpec((1, H), lambda i: (0, 0)),
            pl.BlockSpec((H, 3), lambda i: (0, 0)),
            pl.BlockSpec((1, 3), lambda i: (0, 0)),
            pl.BlockSpec((R, L), lambda i: (i, 0)),
            pl.BlockSpec((R, T - 1), lambda i: (i, 0)),
            pl.BlockSpec((R, K), lambda i: (i, 0)),
        ],
        out_specs=[
            pl.BlockSpec((R, K), lambda i: (i, 0)),
            pl.BlockSpec((R, 1), lambda i: (i, 0)),
        ],
        out_shape=[
            jax.ShapeDtypeStruct((B, K), jnp.float32),
            jax.ShapeDtypeStruct((B, 1), jnp.float32),
        ],
        name="ptrgen_prep",
    )(x, W1, b1_2, W2, b2_2, scores, selfscores, idx_pad)
    mix0T = mix0.T

    m, ssum = pl.pallas_call(
        _stats_body,
        grid=(VP // TVS,),
        in_specs=[
            pl.BlockSpec((DA, B), lambda j: (0, 0)),
            pl.BlockSpec((DA, TVS), lambda j: (0, j)),
        ],
        out_specs=[
            pl.BlockSpec((1, B), lambda j: (0, 0)),
            pl.BlockSpec((1, B), lambda j: (0, 0)),
        ],
        out_shape=[
            jax.ShapeDtypeStruct((1, B), jnp.float32),
            jax.ShapeDtypeStruct((1, B), jnp.float32),
        ],
        name="ptrgen_stats",
    )(xbT, wb)

    z = pl.pallas_call(
        _emit_body,
        grid=(NV,),
        in_specs=[
            pl.BlockSpec((DA, B), lambda j: (0, 0)),
            pl.BlockSpec((DA, TV), lambda j: (0, j)),
            pl.BlockSpec((1, B), lambda j: (0, 0)),
            pl.BlockSpec((1, B), lambda j: (0, 0)),
            pl.BlockSpec((1, B), lambda j: (0, 0)),
        ],
        out_specs=pl.BlockSpec((TVd8, 8, 8, 128), lambda j: (j, 0, 0, 0)),
        out_shape=jax.ShapeDtypeStruct((V // 8, 8, 8, 128), jnp.float32),
        name="ptrgen_emit",
    )(xbT, wb, m, ssum, mix0T)

    sval_w = sval.reshape(NW, NST, SW)
    sc_scatter = pl.kernel(
        _sc_body,
        out_type=(),
        mesh=plsc.VectorSubcoreMesh(core_axis_name="c", subcore_axis_name="s"),
        scratch_types=(
            [pltpu.VMEM((SW,), jnp.int32) for _ in range(NST)]
            + [pltpu.VMEM((SW,), jnp.float32) for _ in range(NST)]
            + [pltpu.VMEM((NST, SW), jnp.float32),
               pltpu.SemaphoreType.DMA,
               pltpu.SemaphoreType.DMA]
        ),
        name="ptrgen_scatter",
    )
    out_ref = jax.new_ref(z.reshape(B * V))
    sc_scatter(out_ref, fidx, sval_w)
    zf = out_ref[...]
    return zf.reshape(V // 8, 8, 8, 128).transpose(1, 3, 0, 2).reshape(B, V)
